# Initial kernel scaffold; baseline (speedup 1.0000x reference)
import jax
import jax.numpy as jnp
from jax.experimental import pallas as pl


def kernel(x, edge_index, W1, b1, W2, b2):
    raise NotImplementedError("write your pallas kernel here")


# speedup vs baseline: 63.2217x; 63.2217x over previous
import functools

import jax
import jax.numpy as jnp
from jax import lax
from jax.experimental import pallas as pl
from jax.experimental.pallas import tpu as pltpu
from jax.experimental.pallas import tpu_sc as plsc

N_NODES = 100000
N_PAD = 100096
STRIPE = N_PAD // 16
CH = STRIPE // 8
EB = 128
U = 8
NW = 32
BN = 5888
GRID = N_PAD // BN

_mesh = plsc.VectorSubcoreMesh(core_axis_name="c", subcore_axis_name="s")


def _make_deg(n_rows2d):
    rows_w = n_rows2d // NW
    g_per_w = rows_w // U

    @functools.partial(
        pl.kernel,
        out_type=jax.ShapeDtypeStruct((2 * N_PAD,), jnp.float32),
        mesh=_mesh,
        compiler_params=pltpu.CompilerParams(use_tc_tiling_on_sc=False),
        scratch_types=[
            pltpu.VMEM_SHARED((N_PAD,), jnp.float32),
            pltpu.VMEM((STRIPE,), jnp.float32),
            pltpu.VMEM((STRIPE,), jnp.float32),
            pltpu.VMEM((U, EB), jnp.int32),
            pltpu.VMEM((EB,), jnp.float32),
        ],
    )
    def deg_kernel(dstp, out, deg_sh, zbuf, zbuf2, didx, ones_v):
        c = lax.axis_index("c")
        s = lax.axis_index("s")
        wid = c * 16 + s
        z = jnp.zeros((16,), jnp.float32)
        one = jnp.ones((16,), jnp.float32)

        def zbody(i, carry):
            zbuf[pl.ds(i * 16, 16)] = z
            return carry

        lax.fori_loop(0, STRIPE // 16, zbody, 0)
        for k in range(EB // 16):
            ones_v[pl.ds(k * 16, 16)] = one
        pltpu.sync_copy(zbuf, deg_sh.at[pl.ds(s * STRIPE, STRIPE)])
        plsc.subcore_barrier()

        def body(g, carry):
            pltpu.sync_copy(dstp.at[pl.ds(wid * rows_w + g * U, U)], didx)
            for j in range(U):
                pltpu.sync_copy(ones_v, deg_sh.at[didx.at[j]], add=True)
            return carry

        lax.fori_loop(0, g_per_w, body, 0)
        plsc.subcore_barrier()
        pltpu.sync_copy(deg_sh.at[pl.ds(s * STRIPE, STRIPE)], zbuf)

        def cbody(i, carry):
            zbuf2[pl.ds(i * 16, 16)] = zbuf[pl.ds(i * 16, 16)]
            return carry

        lax.fori_loop(0, STRIPE // 16, cbody, 0)
        pltpu.sync_copy(zbuf2, out.at[pl.ds(c * N_PAD + s * STRIPE, STRIPE)])

    return deg_kernel


def _make_agg(n_rows2d):
    rows_w = n_rows2d // NW
    g_per_w = rows_w // U

    @functools.partial(
        pl.kernel,
        out_type=jax.ShapeDtypeStruct((2, N_PAD, 16), jnp.float32),
        mesh=_mesh,
        compiler_params=pltpu.CompilerParams(use_tc_tiling_on_sc=False),
        scratch_types=[
            pltpu.VMEM_SHARED((N_PAD, 16), jnp.float32),
            pltpu.VMEM((128, 16), jnp.float32),
            pltpu.VMEM((U, EB), jnp.int32),
            pltpu.VMEM((U, EB), jnp.int32),
            pltpu.VMEM((U, EB, 16), jnp.float32),
            pltpu.SemaphoreType.DMA,
        ],
    )
    def agg_kernel(table, srcp, dstp, out, acc_sh, stage, sidx, didx, rows, sem):
        c = lax.axis_index("c")
        s = lax.axis_index("s")
        wid = c * 16 + s
        z = jnp.zeros((16,), jnp.float32)

        def zrow(i, carry):
            stage[i, :] = z
            return carry

        lax.fori_loop(0, 128, zrow, 0)

        n_tiles = N_PAD // 128
        q_max = -(-n_tiles // 16)

        def ibody(q, carry):
            t = s + q * 16

            @pl.when(t < n_tiles)
            def _():
                pltpu.sync_copy(stage, acc_sh.at[pl.ds(t * 128, 128)])
            return carry

        lax.fori_loop(0, q_max, ibody, 0)
        plsc.subcore_barrier()

        def body(g, carry):
            row = wid * rows_w + g * U
            pltpu.sync_copy(srcp.at[pl.ds(row, U)], sidx)
            pltpu.sync_copy(dstp.at[pl.ds(row, U)], didx)
            cps = [pltpu.async_copy(table.at[sidx.at[j]], rows.at[j], sem)
                   for j in range(U)]
            for cp in cps:
                cp.wait()
            for j in range(U):
                pltpu.sync_copy(rows.at[j], acc_sh.at[didx.at[j]], add=True)
            return carry

        lax.fori_loop(0, g_per_w, body, 0)
        plsc.subcore_barrier()

        def obody(q, carry):
            t = s + q * 16

            @pl.when(t < n_tiles)
            def _():
                pltpu.sync_copy(acc_sh.at[pl.ds(t * 128, 128)], stage)
                pltpu.sync_copy(stage, out.at[c, pl.ds(t * 128, 128)])
            return carry

        lax.fori_loop(0, q_max, obody, 0)

    return agg_kernel


def _tc_scale1(degs, xpad, W1):
    def body(deg_ref, x_ref, w_ref, dis_ref, h_ref):
        deg = jnp.sum(deg_ref[...], axis=0) + 1.0
        dis = lax.rsqrt(deg)[:, None]
        h = jnp.dot(x_ref[...], w_ref[...], preferred_element_type=jnp.float32)
        dis_ref[...] = dis
        h_ref[...] = h * dis

    return pl.pallas_call(
        body,
        grid=(GRID,),
        in_specs=[
            pl.BlockSpec((2, BN), lambda i: (0, i)),
            pl.BlockSpec((BN, 8), lambda i: (i, 0)),
            pl.BlockSpec((8, 16), lambda i: (0, 0)),
        ],
        out_specs=[
            pl.BlockSpec((BN, 1), lambda i: (i, 0)),
            pl.BlockSpec((BN, 16), lambda i: (i, 0)),
        ],
        out_shape=[
            jax.ShapeDtypeStruct((N_PAD, 1), jnp.float32),
            jax.ShapeDtypeStruct((N_PAD, 16), jnp.float32),
        ],
    )(degs, xpad, W1)


def _tc_mid(acc, h1p, dis, b1, W2p):
    def body(a_ref, h_ref, dis_ref, b_ref, w_ref, o_ref):
        dis = dis_ref[...]
        p = (a_ref[0] + a_ref[1] + h_ref[...]) * dis + b_ref[...]
        r = jnp.maximum(p, 0.0)
        o_ref[...] = jnp.dot(r, w_ref[...],
                             preferred_element_type=jnp.float32) * dis

    return pl.pallas_call(
        body,
        grid=(GRID,),
        in_specs=[
            pl.BlockSpec((2, BN, 16), lambda i: (0, i, 0)),
            pl.BlockSpec((BN, 16), lambda i: (i, 0)),
            pl.BlockSpec((BN, 1), lambda i: (i, 0)),
            pl.BlockSpec((1, 16), lambda i: (0, 0)),
            pl.BlockSpec((16, 16), lambda i: (0, 0)),
        ],
        out_specs=pl.BlockSpec((BN, 16), lambda i: (i, 0)),
        out_shape=jax.ShapeDtypeStruct((N_PAD, 16), jnp.float32),
    )(acc, h1p, dis, b1, W2p)


def _tc_final(acc, h2p, dis, b2p):
    def body(a_ref, h_ref, dis_ref, b_ref, o_ref):
        o_ref[...] = jax.nn.sigmoid(
            (a_ref[0] + a_ref[1] + h_ref[...]) * dis_ref[...] + b_ref[...])

    return pl.pallas_call(
        body,
        grid=(GRID,),
        in_specs=[
            pl.BlockSpec((2, BN, 16), lambda i: (0, i, 0)),
            pl.BlockSpec((BN, 16), lambda i: (i, 0)),
            pl.BlockSpec((BN, 1), lambda i: (i, 0)),
            pl.BlockSpec((1, 16), lambda i: (0, 0)),
        ],
        out_specs=pl.BlockSpec((BN, 16), lambda i: (i, 0)),
        out_shape=jax.ShapeDtypeStruct((N_PAD, 16), jnp.float32),
    )(acc, h2p, dis, b2p)


def kernel(x, edge_index, W1, b1, W2, b2):
    n_edges = edge_index.shape[1]
    chunk = NW * EB * U
    epad = -(-n_edges // chunk) * chunk
    pad = epad - n_edges

    src = edge_index[0].astype(jnp.int32)
    dst = edge_index[1].astype(jnp.int32)
    junk = jnp.full((pad,), N_NODES, jnp.int32)
    srcp = jnp.concatenate([src, junk]).reshape(-1, EB)
    dstp = jnp.concatenate([dst, junk]).reshape(-1, EB)
    n_rows2d = srcp.shape[0]

    xpad = jnp.pad(x, ((0, N_PAD - N_NODES), (0, 0)))
    W2p = jnp.pad(W2, ((0, 0), (0, 16 - W2.shape[1])))
    b1r = b1.reshape(1, 16)
    b2p = jnp.pad(b2, (0, 16 - b2.shape[0])).reshape(1, 16)

    degs = _make_deg(n_rows2d)(dstp).reshape(2, N_PAD)
    dis, h1p = _tc_scale1(degs, xpad, W1)

    agg = _make_agg(n_rows2d)
    acc1 = agg(h1p, srcp, dstp)
    h2p = _tc_mid(acc1, h1p, dis, b1r, W2p)
    acc2 = agg(h2p, srcp, dstp)
    out2 = _tc_final(acc2, h2p, dis, b2p)
    return out2[:N_NODES, :W2.shape[1]]


# speedup vs baseline: 67.0999x; 1.0613x over previous
import functools

import jax
import jax.numpy as jnp
from jax import lax
from jax.experimental import pallas as pl
from jax.experimental.pallas import tpu as pltpu
from jax.experimental.pallas import tpu_sc as plsc

N_NODES = 100000
N_PAD = 100096
STRIPE = N_PAD // 16
CH = STRIPE // 8
EB = 128
U = 8
NW = 32
BN = 5888
GRID = N_PAD // BN

_mesh = plsc.VectorSubcoreMesh(core_axis_name="c", subcore_axis_name="s")


def _make_deg(n_rows2d):
    rows_w = n_rows2d // NW
    g_per_w = rows_w // U

    @functools.partial(
        pl.kernel,
        out_type=jax.ShapeDtypeStruct((2 * N_PAD,), jnp.float32),
        mesh=_mesh,
        compiler_params=pltpu.CompilerParams(use_tc_tiling_on_sc=False),
        scratch_types=[
            pltpu.VMEM_SHARED((N_PAD,), jnp.float32),
            pltpu.VMEM((STRIPE,), jnp.float32),
            pltpu.VMEM((STRIPE,), jnp.float32),
            pltpu.VMEM((U, EB), jnp.int32),
            pltpu.VMEM((EB,), jnp.float32),
        ],
    )
    def deg_kernel(dstp, out, deg_sh, zbuf, zbuf2, didx, ones_v):
        c = lax.axis_index("c")
        s = lax.axis_index("s")
        wid = c * 16 + s
        z = jnp.zeros((16,), jnp.float32)
        one = jnp.ones((16,), jnp.float32)

        def zbody(i, carry):
            zbuf[pl.ds(i * 16, 16)] = z
            return carry

        lax.fori_loop(0, STRIPE // 16, zbody, 0)
        for k in range(EB // 16):
            ones_v[pl.ds(k * 16, 16)] = one
        pltpu.sync_copy(zbuf, deg_sh.at[pl.ds(s * STRIPE, STRIPE)])
        plsc.subcore_barrier()

        def body(g, carry):
            pltpu.sync_copy(dstp.at[pl.ds(wid * rows_w + g * U, U)], didx)
            for j in range(U):
                pltpu.sync_copy(ones_v, deg_sh.at[didx.at[j]], add=True)
            return carry

        lax.fori_loop(0, g_per_w, body, 0)
        plsc.subcore_barrier()
        pltpu.sync_copy(deg_sh.at[pl.ds(s * STRIPE, STRIPE)], zbuf)

        def cbody(i, carry):
            zbuf2[pl.ds(i * 16, 16)] = zbuf[pl.ds(i * 16, 16)]
            return carry

        lax.fori_loop(0, STRIPE // 16, cbody, 0)
        pltpu.sync_copy(zbuf2, out.at[pl.ds(c * N_PAD + s * STRIPE, STRIPE)])

    return deg_kernel


def _make_agg(n_rows2d):
    rows_w = n_rows2d // NW
    g_per_w = rows_w // U

    @functools.partial(
        pl.kernel,
        out_type=jax.ShapeDtypeStruct((2, N_PAD, 16), jnp.float32),
        mesh=_mesh,
        compiler_params=pltpu.CompilerParams(use_tc_tiling_on_sc=False),
        scratch_types=[
            pltpu.VMEM_SHARED((N_PAD, 16), jnp.float32),
            pltpu.VMEM((128, 16), jnp.float32),
            pltpu.VMEM((U, EB), jnp.int32),
            pltpu.VMEM((U, EB), jnp.int32),
            pltpu.VMEM((U, EB, 16), jnp.float32),
            pltpu.SemaphoreType.DMA,
            pltpu.SemaphoreType.DMA,
        ],
    )
    def agg_kernel(table, srcp, dstp, out, acc_sh, stage, sidx, didx, rows,
                   sem, sem2):
        c = lax.axis_index("c")
        s = lax.axis_index("s")
        wid = c * 16 + s
        z = jnp.zeros((16,), jnp.float32)

        def zrow(i, carry):
            stage[i, :] = z
            return carry

        lax.fori_loop(0, 128, zrow, 0)

        n_tiles = N_PAD // 128
        q_max = -(-n_tiles // 16)

        def ibody(q, carry):
            t = s + q * 16

            @pl.when(t < n_tiles)
            def _():
                pltpu.sync_copy(stage, acc_sh.at[pl.ds(t * 128, 128)])
            return carry

        lax.fori_loop(0, q_max, ibody, 0)
        plsc.subcore_barrier()

        def body(g, carry):
            row = wid * rows_w + g * U
            pltpu.sync_copy(srcp.at[pl.ds(row, U)], sidx)
            pltpu.sync_copy(dstp.at[pl.ds(row, U)], didx)
            cps = [pltpu.async_copy(table.at[sidx.at[j]], rows.at[j], sem)
                   for j in range(U)]
            for cp in cps:
                cp.wait()
            cps2 = [pltpu.async_copy(rows.at[j], acc_sh.at[didx.at[j]], sem2,
                                     add=True)
                    for j in range(U)]
            for cp in cps2:
                cp.wait()
            return carry

        lax.fori_loop(0, g_per_w, body, 0)
        plsc.subcore_barrier()

        def obody(q, carry):
            t = s + q * 16

            @pl.when(t < n_tiles)
            def _():
                pltpu.sync_copy(acc_sh.at[pl.ds(t * 128, 128)], stage)
                pltpu.sync_copy(stage, out.at[c, pl.ds(t * 128, 128)])
            return carry

        lax.fori_loop(0, q_max, obody, 0)

    return agg_kernel


def _tc_scale1(degs, xpad, W1):
    def body(deg_ref, x_ref, w_ref, dis_ref, h_ref):
        deg = jnp.sum(deg_ref[...], axis=0) + 1.0
        dis = lax.rsqrt(deg)[:, None]
        h = jnp.dot(x_ref[...], w_ref[...], preferred_element_type=jnp.float32)
        dis_ref[...] = dis
        h_ref[...] = h * dis

    return pl.pallas_call(
        body,
        grid=(GRID,),
        in_specs=[
            pl.BlockSpec((2, BN), lambda i: (0, i)),
            pl.BlockSpec((BN, 8), lambda i: (i, 0)),
            pl.BlockSpec((8, 16), lambda i: (0, 0)),
        ],
        out_specs=[
            pl.BlockSpec((BN, 1), lambda i: (i, 0)),
            pl.BlockSpec((BN, 16), lambda i: (i, 0)),
        ],
        out_shape=[
            jax.ShapeDtypeStruct((N_PAD, 1), jnp.float32),
            jax.ShapeDtypeStruct((N_PAD, 16), jnp.float32),
        ],
    )(degs, xpad, W1)


def _tc_mid(acc, h1p, dis, b1, W2p):
    def body(a_ref, h_ref, dis_ref, b_ref, w_ref, o_ref):
        dis = dis_ref[...]
        p = (a_ref[0] + a_ref[1] + h_ref[...]) * dis + b_ref[...]
        r = jnp.maximum(p, 0.0)
        o_ref[...] = jnp.dot(r, w_ref[...],
                             preferred_element_type=jnp.float32) * dis

    return pl.pallas_call(
        body,
        grid=(GRID,),
        in_specs=[
            pl.BlockSpec((2, BN, 16), lambda i: (0, i, 0)),
            pl.BlockSpec((BN, 16), lambda i: (i, 0)),
            pl.BlockSpec((BN, 1), lambda i: (i, 0)),
            pl.BlockSpec((1, 16), lambda i: (0, 0)),
            pl.BlockSpec((16, 16), lambda i: (0, 0)),
        ],
        out_specs=pl.BlockSpec((BN, 16), lambda i: (i, 0)),
        out_shape=jax.ShapeDtypeStruct((N_PAD, 16), jnp.float32),
    )(acc, h1p, dis, b1, W2p)


def _tc_final(acc, h2p, dis, b2p):
    def body(a_ref, h_ref, dis_ref, b_ref, o_ref):
        o_ref[...] = jax.nn.sigmoid(
            (a_ref[0] + a_ref[1] + h_ref[...]) * dis_ref[...] + b_ref[...])

    return pl.pallas_call(
        body,
        grid=(GRID,),
        in_specs=[
            pl.BlockSpec((2, BN, 16), lambda i: (0, i, 0)),
            pl.BlockSpec((BN, 16), lambda i: (i, 0)),
            pl.BlockSpec((BN, 1), lambda i: (i, 0)),
            pl.BlockSpec((1, 16), lambda i: (0, 0)),
        ],
        out_specs=pl.BlockSpec((BN, 16), lambda i: (i, 0)),
        out_shape=jax.ShapeDtypeStruct((N_PAD, 16), jnp.float32),
    )(acc, h2p, dis, b2p)


def kernel(x, edge_index, W1, b1, W2, b2):
    n_edges = edge_index.shape[1]
    chunk = NW * EB * U
    epad = -(-n_edges // chunk) * chunk
    pad = epad - n_edges

    src = edge_index[0].astype(jnp.int32)
    dst = edge_index[1].astype(jnp.int32)
    junk = jnp.full((pad,), N_NODES, jnp.int32)
    srcp = jnp.concatenate([src, junk]).reshape(-1, EB)
    dstp = jnp.concatenate([dst, junk]).reshape(-1, EB)
    n_rows2d = srcp.shape[0]

    xpad = jnp.pad(x, ((0, N_PAD - N_NODES), (0, 0)))
    W2p = jnp.pad(W2, ((0, 0), (0, 16 - W2.shape[1])))
    b1r = b1.reshape(1, 16)
    b2p = jnp.pad(b2, (0, 16 - b2.shape[0])).reshape(1, 16)

    degs = _make_deg(n_rows2d)(dstp).reshape(2, N_PAD)
    dis, h1p = _tc_scale1(degs, xpad, W1)

    agg = _make_agg(n_rows2d)
    acc1 = agg(h1p, srcp, dstp)
    h2p = _tc_mid(acc1, h1p, dis, b1r, W2p)
    acc2 = agg(h2p, srcp, dstp)
    out2 = _tc_final(acc2, h2p, dis, b2p)
    return out2[:N_NODES, :W2.shape[1]]


# speedup vs baseline: 71.2666x; 1.0621x over previous
import functools

import jax
import jax.numpy as jnp
from jax import lax
from jax.experimental import pallas as pl
from jax.experimental.pallas import tpu as pltpu
from jax.experimental.pallas import tpu_sc as plsc

N_NODES = 100000
N_PAD = 100096
STRIPE = N_PAD // 16
CH = STRIPE // 8
EB = 128
U = 12
NW = 32
BN = 5888
GRID = N_PAD // BN

_mesh = plsc.VectorSubcoreMesh(core_axis_name="c", subcore_axis_name="s")


def _make_deg(n_rows2d):
    rows_w = n_rows2d // NW
    g_per_w = rows_w // U

    @functools.partial(
        pl.kernel,
        out_type=jax.ShapeDtypeStruct((2 * N_PAD,), jnp.float32),
        mesh=_mesh,
        compiler_params=pltpu.CompilerParams(use_tc_tiling_on_sc=False),
        scratch_types=[
            pltpu.VMEM_SHARED((N_PAD,), jnp.float32),
            pltpu.VMEM((STRIPE,), jnp.float32),
            pltpu.VMEM((STRIPE,), jnp.float32),
            pltpu.VMEM((U, EB), jnp.int32),
            pltpu.VMEM((EB,), jnp.float32),
        ],
    )
    def deg_kernel(dstp, out, deg_sh, zbuf, zbuf2, didx, ones_v):
        c = lax.axis_index("c")
        s = lax.axis_index("s")
        wid = c * 16 + s
        z = jnp.zeros((16,), jnp.float32)
        one = jnp.ones((16,), jnp.float32)

        def zbody(i, carry):
            zbuf[pl.ds(i * 16, 16)] = z
            return carry

        lax.fori_loop(0, STRIPE // 16, zbody, 0)
        for k in range(EB // 16):
            ones_v[pl.ds(k * 16, 16)] = one
        pltpu.sync_copy(zbuf, deg_sh.at[pl.ds(s * STRIPE, STRIPE)])
        plsc.subcore_barrier()

        def body(g, carry):
            pltpu.sync_copy(dstp.at[pl.ds(wid * rows_w + g * U, U)], didx)
            for j in range(U):
                pltpu.sync_copy(ones_v, deg_sh.at[didx.at[j]], add=True)
            return carry

        lax.fori_loop(0, g_per_w, body, 0)
        plsc.subcore_barrier()
        pltpu.sync_copy(deg_sh.at[pl.ds(s * STRIPE, STRIPE)], zbuf)

        def cbody(i, carry):
            zbuf2[pl.ds(i * 16, 16)] = zbuf[pl.ds(i * 16, 16)]
            return carry

        lax.fori_loop(0, STRIPE // 16, cbody, 0)
        pltpu.sync_copy(zbuf2, out.at[pl.ds(c * N_PAD + s * STRIPE, STRIPE)])

    return deg_kernel


def _make_agg(n_rows2d):
    rows_w = n_rows2d // NW
    g_per_w = rows_w // U

    @functools.partial(
        pl.kernel,
        out_type=jax.ShapeDtypeStruct((2, N_PAD, 16), jnp.float32),
        mesh=_mesh,
        compiler_params=pltpu.CompilerParams(use_tc_tiling_on_sc=False),
        scratch_types=[
            pltpu.VMEM_SHARED((N_PAD, 16), jnp.float32),
            pltpu.VMEM((128, 16), jnp.float32),
            pltpu.VMEM((U, EB), jnp.int32),
            pltpu.VMEM((U, EB), jnp.int32),
            pltpu.VMEM((U, EB, 16), jnp.float32),
            pltpu.SemaphoreType.DMA,
            pltpu.SemaphoreType.DMA,
        ],
    )
    def agg_kernel(table, srcp, dstp, out, acc_sh, stage, sidx, didx, rows,
                   sem, sem2):
        c = lax.axis_index("c")
        s = lax.axis_index("s")
        wid = c * 16 + s
        z = jnp.zeros((16,), jnp.float32)

        def zrow(i, carry):
            stage[i, :] = z
            return carry

        lax.fori_loop(0, 128, zrow, 0)

        n_tiles = N_PAD // 128
        q_max = -(-n_tiles // 16)

        def ibody(q, carry):
            t = s + q * 16

            @pl.when(t < n_tiles)
            def _():
                pltpu.sync_copy(stage, acc_sh.at[pl.ds(t * 128, 128)])
            return carry

        lax.fori_loop(0, q_max, ibody, 0)
        plsc.subcore_barrier()

        def body(g, carry):
            row = wid * rows_w + g * U
            ci = [pltpu.async_copy(srcp.at[pl.ds(row, U)], sidx, sem2),
                  pltpu.async_copy(dstp.at[pl.ds(row, U)], didx, sem2)]
            for cp in ci:
                cp.wait()
            cps = [pltpu.async_copy(table.at[sidx.at[j]], rows.at[j], sem)
                   for j in range(U)]
            for cp in cps:
                cp.wait()
            cps2 = [pltpu.async_copy(rows.at[j], acc_sh.at[didx.at[j]], sem2,
                                     add=True)
                    for j in range(U)]
            for cp in cps2:
                cp.wait()
            return carry

        lax.fori_loop(0, g_per_w, body, 0)
        plsc.subcore_barrier()

        def obody(q, carry):
            t = s + q * 16

            @pl.when(t < n_tiles)
            def _():
                pltpu.sync_copy(acc_sh.at[pl.ds(t * 128, 128)], stage)
                pltpu.sync_copy(stage, out.at[c, pl.ds(t * 128, 128)])
            return carry

        lax.fori_loop(0, q_max, obody, 0)

    return agg_kernel


def _tc_scale1(degs, xpad, W1):
    def body(deg_ref, x_ref, w_ref, dis_ref, h_ref):
        deg = jnp.sum(deg_ref[...], axis=0) + 1.0
        dis = lax.rsqrt(deg)[:, None]
        h = jnp.dot(x_ref[...], w_ref[...], preferred_element_type=jnp.float32)
        dis_ref[...] = dis
        h_ref[...] = h * dis

    return pl.pallas_call(
        body,
        grid=(GRID,),
        in_specs=[
            pl.BlockSpec((2, BN), lambda i: (0, i)),
            pl.BlockSpec((BN, 8), lambda i: (i, 0)),
            pl.BlockSpec((8, 16), lambda i: (0, 0)),
        ],
        out_specs=[
            pl.BlockSpec((BN, 1), lambda i: (i, 0)),
            pl.BlockSpec((BN, 16), lambda i: (i, 0)),
        ],
        out_shape=[
            jax.ShapeDtypeStruct((N_PAD, 1), jnp.float32),
            jax.ShapeDtypeStruct((N_PAD, 16), jnp.float32),
        ],
    )(degs, xpad, W1)


def _tc_mid(acc, h1p, dis, b1, W2p):
    def body(a_ref, h_ref, dis_ref, b_ref, w_ref, o_ref):
        dis = dis_ref[...]
        p = (a_ref[0] + a_ref[1] + h_ref[...]) * dis + b_ref[...]
        r = jnp.maximum(p, 0.0)
        o_ref[...] = jnp.dot(r, w_ref[...],
                             preferred_element_type=jnp.float32) * dis

    return pl.pallas_call(
        body,
        grid=(GRID,),
        in_specs=[
            pl.BlockSpec((2, BN, 16), lambda i: (0, i, 0)),
            pl.BlockSpec((BN, 16), lambda i: (i, 0)),
            pl.BlockSpec((BN, 1), lambda i: (i, 0)),
            pl.BlockSpec((1, 16), lambda i: (0, 0)),
            pl.BlockSpec((16, 16), lambda i: (0, 0)),
        ],
        out_specs=pl.BlockSpec((BN, 16), lambda i: (i, 0)),
        out_shape=jax.ShapeDtypeStruct((N_PAD, 16), jnp.float32),
    )(acc, h1p, dis, b1, W2p)


def _tc_final(acc, h2p, dis, b2p):
    def body(a_ref, h_ref, dis_ref, b_ref, o_ref):
        o_ref[...] = jax.nn.sigmoid(
            (a_ref[0] + a_ref[1] + h_ref[...]) * dis_ref[...] + b_ref[...])

    return pl.pallas_call(
        body,
        grid=(GRID,),
        in_specs=[
            pl.BlockSpec((2, BN, 16), lambda i: (0, i, 0)),
            pl.BlockSpec((BN, 16), lambda i: (i, 0)),
            pl.BlockSpec((BN, 1), lambda i: (i, 0)),
            pl.BlockSpec((1, 16), lambda i: (0, 0)),
        ],
        out_specs=pl.BlockSpec((BN, 16), lambda i: (i, 0)),
        out_shape=jax.ShapeDtypeStruct((N_PAD, 16), jnp.float32),
    )(acc, h2p, dis, b2p)


def kernel(x, edge_index, W1, b1, W2, b2):
    n_edges = edge_index.shape[1]
    chunk = NW * EB * U
    epad = -(-n_edges // chunk) * chunk
    pad = epad - n_edges

    src = edge_index[0].astype(jnp.int32)
    dst = edge_index[1].astype(jnp.int32)
    junk = jnp.full((pad,), N_NODES, jnp.int32)
    srcp = jnp.concatenate([src, junk]).reshape(-1, EB)
    dstp = jnp.concatenate([dst, junk]).reshape(-1, EB)
    n_rows2d = srcp.shape[0]

    xpad = jnp.pad(x, ((0, N_PAD - N_NODES), (0, 0)))
    W2p = jnp.pad(W2, ((0, 0), (0, 16 - W2.shape[1])))
    b1r = b1.reshape(1, 16)
    b2p = jnp.pad(b2, (0, 16 - b2.shape[0])).reshape(1, 16)

    degs = _make_deg(n_rows2d)(dstp).reshape(2, N_PAD)
    dis, h1p = _tc_scale1(degs, xpad, W1)

    agg = _make_agg(n_rows2d)
    acc1 = agg(h1p, srcp, dstp)
    h2p = _tc_mid(acc1, h1p, dis, b1r, W2p)
    acc2 = agg(h2p, srcp, dstp)
    out2 = _tc_final(acc2, h2p, dis, b2p)
    return out2[:N_NODES, :W2.shape[1]]


# speedup vs baseline: 75.3259x; 1.0570x over previous
import functools

import jax
import jax.numpy as jnp
from jax import lax
from jax.experimental import pallas as pl
from jax.experimental.pallas import tpu as pltpu
from jax.experimental.pallas import tpu_sc as plsc

N_NODES = 100000
N_PAD = 100096
STRIPE = N_PAD // 16
CH = STRIPE // 8
EB = 128
U = 12
NW = 32
BN = 5888
GRID = N_PAD // BN

_mesh = plsc.VectorSubcoreMesh(core_axis_name="c", subcore_axis_name="s")


def _make_deg(n_rows2d):
    rows_w = n_rows2d // NW
    g_per_w = rows_w // U

    @functools.partial(
        pl.kernel,
        out_type=jax.ShapeDtypeStruct((2 * N_PAD,), jnp.float32),
        mesh=_mesh,
        compiler_params=pltpu.CompilerParams(use_tc_tiling_on_sc=False),
        scratch_types=[
            pltpu.VMEM_SHARED((N_PAD,), jnp.float32),
            pltpu.VMEM((STRIPE,), jnp.float32),
            pltpu.VMEM((STRIPE,), jnp.float32),
            pltpu.VMEM((U, EB), jnp.int32),
            pltpu.VMEM((EB,), jnp.float32),
        ],
    )
    def deg_kernel(dstp, out, deg_sh, zbuf, zbuf2, didx, ones_v):
        c = lax.axis_index("c")
        s = lax.axis_index("s")
        wid = c * 16 + s
        z = jnp.zeros((16,), jnp.float32)
        one = jnp.ones((16,), jnp.float32)

        def zbody(i, carry):
            zbuf[pl.ds(i * 16, 16)] = z
            return carry

        lax.fori_loop(0, STRIPE // 16, zbody, 0)
        for k in range(EB // 16):
            ones_v[pl.ds(k * 16, 16)] = one
        pltpu.sync_copy(zbuf, deg_sh.at[pl.ds(s * STRIPE, STRIPE)])
        plsc.subcore_barrier()

        def body(g, carry):
            pltpu.sync_copy(dstp.at[pl.ds(wid * rows_w + g * U, U)], didx)
            for j in range(U):
                pltpu.sync_copy(ones_v, deg_sh.at[didx.at[j]], add=True)
            return carry

        lax.fori_loop(0, g_per_w, body, 0)
        plsc.subcore_barrier()
        pltpu.sync_copy(deg_sh.at[pl.ds(s * STRIPE, STRIPE)], zbuf)

        def cbody(i, carry):
            zbuf2[pl.ds(i * 16, 16)] = zbuf[pl.ds(i * 16, 16)]
            return carry

        lax.fori_loop(0, STRIPE // 16, cbody, 0)
        pltpu.sync_copy(zbuf2, out.at[pl.ds(c * N_PAD + s * STRIPE, STRIPE)])

    return deg_kernel


def _make_agg(n_rows2d):
    rows_w = n_rows2d // NW
    g_per_w = rows_w // U

    @functools.partial(
        pl.kernel,
        out_type=jax.ShapeDtypeStruct((2, N_PAD, 16), jnp.float32),
        mesh=_mesh,
        compiler_params=pltpu.CompilerParams(use_tc_tiling_on_sc=False),
        scratch_types=[
            pltpu.VMEM_SHARED((N_PAD, 16), jnp.float32),
            pltpu.VMEM((128, 16), jnp.float32),
            pltpu.VMEM((U, EB), jnp.int32),
            pltpu.VMEM((U, EB), jnp.int32),
            pltpu.VMEM((U, EB, 16), jnp.float32),
            pltpu.SemaphoreType.DMA,
            pltpu.SemaphoreType.DMA,
        ],
    )
    def agg_kernel(table, srcp, dstp, out, acc_sh, stage, sidx, didx, rows,
                   sem, sem2):
        c = lax.axis_index("c")
        s = lax.axis_index("s")
        wid = c * 16 + s
        z = jnp.zeros((16,), jnp.float32)

        def zrow(i, carry):
            stage[i, :] = z
            return carry

        lax.fori_loop(0, 128, zrow, 0)

        n_tiles = N_PAD // 128
        q_max = -(-n_tiles // 16)

        def ibody(q, carry):
            t = s + q * 16

            @pl.when(t < n_tiles)
            def _():
                pltpu.sync_copy(stage, acc_sh.at[pl.ds(t * 128, 128)])
            return carry

        lax.fori_loop(0, q_max, ibody, 0)
        plsc.subcore_barrier()

        def body(g, carry):
            row = wid * rows_w + g * U
            ci = [pltpu.async_copy(srcp.at[pl.ds(row, U)], sidx, sem2),
                  pltpu.async_copy(dstp.at[pl.ds(row, U)], didx, sem2)]
            for cp in ci:
                cp.wait()
            h = U // 2
            cpa = [pltpu.async_copy(table.at[sidx.at[j]], rows.at[j], sem)
                   for j in range(h)]
            cpb = [pltpu.async_copy(table.at[sidx.at[j]], rows.at[j], sem)
                   for j in range(h, U)]
            for cp in cpa:
                cp.wait()
            csa = [pltpu.async_copy(rows.at[j], acc_sh.at[didx.at[j]], sem2,
                                    add=True)
                   for j in range(h)]
            for cp in cpb:
                cp.wait()
            csb = [pltpu.async_copy(rows.at[j], acc_sh.at[didx.at[j]], sem2,
                                    add=True)
                   for j in range(h, U)]
            for cp in csa + csb:
                cp.wait()
            return carry

        lax.fori_loop(0, g_per_w, body, 0)
        plsc.subcore_barrier()

        def obody(q, carry):
            t = s + q * 16

            @pl.when(t < n_tiles)
            def _():
                pltpu.sync_copy(acc_sh.at[pl.ds(t * 128, 128)], stage)
                pltpu.sync_copy(stage, out.at[c, pl.ds(t * 128, 128)])
            return carry

        lax.fori_loop(0, q_max, obody, 0)

    return agg_kernel


def _tc_scale1(degs, xpad, W1):
    def body(deg_ref, x_ref, w_ref, dis_ref, h_ref):
        deg = jnp.sum(deg_ref[...], axis=0) + 1.0
        dis = lax.rsqrt(deg)[:, None]
        h = jnp.dot(x_ref[...], w_ref[...], preferred_element_type=jnp.float32)
        dis_ref[...] = dis
        h_ref[...] = h * dis

    return pl.pallas_call(
        body,
        grid=(GRID,),
        in_specs=[
            pl.BlockSpec((2, BN), lambda i: (0, i)),
            pl.BlockSpec((BN, 8), lambda i: (i, 0)),
            pl.BlockSpec((8, 16), lambda i: (0, 0)),
        ],
        out_specs=[
            pl.BlockSpec((BN, 1), lambda i: (i, 0)),
            pl.BlockSpec((BN, 16), lambda i: (i, 0)),
        ],
        out_shape=[
            jax.ShapeDtypeStruct((N_PAD, 1), jnp.float32),
            jax.ShapeDtypeStruct((N_PAD, 16), jnp.float32),
        ],
    )(degs, xpad, W1)


def _tc_mid(acc, h1p, dis, b1, W2p):
    def body(a_ref, h_ref, dis_ref, b_ref, w_ref, o_ref):
        dis = dis_ref[...]
        p = (a_ref[0] + a_ref[1] + h_ref[...]) * dis + b_ref[...]
        r = jnp.maximum(p, 0.0)
        o_ref[...] = jnp.dot(r, w_ref[...],
                             preferred_element_type=jnp.float32) * dis

    return pl.pallas_call(
        body,
        grid=(GRID,),
        in_specs=[
            pl.BlockSpec((2, BN, 16), lambda i: (0, i, 0)),
            pl.BlockSpec((BN, 16), lambda i: (i, 0)),
            pl.BlockSpec((BN, 1), lambda i: (i, 0)),
            pl.BlockSpec((1, 16), lambda i: (0, 0)),
            pl.BlockSpec((16, 16), lambda i: (0, 0)),
        ],
        out_specs=pl.BlockSpec((BN, 16), lambda i: (i, 0)),
        out_shape=jax.ShapeDtypeStruct((N_PAD, 16), jnp.float32),
    )(acc, h1p, dis, b1, W2p)


def _tc_final(acc, h2p, dis, b2p):
    def body(a_ref, h_ref, dis_ref, b_ref, o_ref):
        o_ref[...] = jax.nn.sigmoid(
            (a_ref[0] + a_ref[1] + h_ref[...]) * dis_ref[...] + b_ref[...])

    return pl.pallas_call(
        body,
        grid=(GRID,),
        in_specs=[
            pl.BlockSpec((2, BN, 16), lambda i: (0, i, 0)),
            pl.BlockSpec((BN, 16), lambda i: (i, 0)),
            pl.BlockSpec((BN, 1), lambda i: (i, 0)),
            pl.BlockSpec((1, 16), lambda i: (0, 0)),
        ],
        out_specs=pl.BlockSpec((BN, 16), lambda i: (i, 0)),
        out_shape=jax.ShapeDtypeStruct((N_PAD, 16), jnp.float32),
    )(acc, h2p, dis, b2p)


def kernel(x, edge_index, W1, b1, W2, b2):
    n_edges = edge_index.shape[1]
    chunk = NW * EB * U
    epad = -(-n_edges // chunk) * chunk
    pad = epad - n_edges

    src = edge_index[0].astype(jnp.int32)
    dst = edge_index[1].astype(jnp.int32)
    junk = jnp.full((pad,), N_NODES, jnp.int32)
    srcp = jnp.concatenate([src, junk]).reshape(-1, EB)
    dstp = jnp.concatenate([dst, junk]).reshape(-1, EB)
    n_rows2d = srcp.shape[0]

    xpad = jnp.pad(x, ((0, N_PAD - N_NODES), (0, 0)))
    W2p = jnp.pad(W2, ((0, 0), (0, 16 - W2.shape[1])))
    b1r = b1.reshape(1, 16)
    b2p = jnp.pad(b2, (0, 16 - b2.shape[0])).reshape(1, 16)

    degs = _make_deg(n_rows2d)(dstp).reshape(2, N_PAD)
    dis, h1p = _tc_scale1(degs, xpad, W1)

    agg = _make_agg(n_rows2d)
    acc1 = agg(h1p, srcp, dstp)
    h2p = _tc_mid(acc1, h1p, dis, b1r, W2p)
    acc2 = agg(h2p, srcp, dstp)
    out2 = _tc_final(acc2, h2p, dis, b2p)
    return out2[:N_NODES, :W2.shape[1]]


# speedup vs baseline: 77.6127x; 1.0304x over previous
import functools

import jax
import jax.numpy as jnp
from jax import lax
from jax.experimental import pallas as pl
from jax.experimental.pallas import tpu as pltpu
from jax.experimental.pallas import tpu_sc as plsc

N_NODES = 100000
N_PAD = 100096
STRIPE = N_PAD // 16
CH = STRIPE // 8
EB = 128
U = 12
NW = 32
BN = 5888
GRID = N_PAD // BN

_mesh = plsc.VectorSubcoreMesh(core_axis_name="c", subcore_axis_name="s")


def _make_deg(n_rows2d):
    rows_w = n_rows2d // NW
    g_per_w = rows_w // U

    @functools.partial(
        pl.kernel,
        out_type=jax.ShapeDtypeStruct((2 * N_PAD,), jnp.float32),
        mesh=_mesh,
        compiler_params=pltpu.CompilerParams(use_tc_tiling_on_sc=False),
        scratch_types=[
            pltpu.VMEM_SHARED((N_PAD,), jnp.float32),
            pltpu.VMEM((STRIPE,), jnp.float32),
            pltpu.VMEM((STRIPE,), jnp.float32),
            pltpu.VMEM((U, EB), jnp.int32),
            pltpu.VMEM((EB,), jnp.float32),
            pltpu.SemaphoreType.DMA,
        ],
    )
    def deg_kernel(dstp, out, deg_sh, zbuf, zbuf2, didx, ones_v, sem):
        c = lax.axis_index("c")
        s = lax.axis_index("s")
        wid = c * 16 + s
        z = jnp.zeros((16,), jnp.float32)
        one = jnp.ones((16,), jnp.float32)

        def zbody(i, carry):
            zbuf[pl.ds(i * 16, 16)] = z
            return carry

        lax.fori_loop(0, STRIPE // 16, zbody, 0)
        for k in range(EB // 16):
            ones_v[pl.ds(k * 16, 16)] = one
        pltpu.sync_copy(zbuf, deg_sh.at[pl.ds(s * STRIPE, STRIPE)])
        plsc.subcore_barrier()

        def body(g, carry):
            pltpu.sync_copy(dstp.at[pl.ds(wid * rows_w + g * U, U)], didx)
            cps = [pltpu.async_copy(ones_v, deg_sh.at[didx.at[j]], sem,
                                    add=True)
                   for j in range(U)]
            for cp in cps:
                cp.wait()
            return carry

        lax.fori_loop(0, g_per_w, body, 0)
        plsc.subcore_barrier()
        pltpu.sync_copy(deg_sh.at[pl.ds(s * STRIPE, STRIPE)], zbuf)

        def cbody(i, carry):
            zbuf2[pl.ds(i * 16, 16)] = zbuf[pl.ds(i * 16, 16)]
            return carry

        lax.fori_loop(0, STRIPE // 16, cbody, 0)
        pltpu.sync_copy(zbuf2, out.at[pl.ds(c * N_PAD + s * STRIPE, STRIPE)])

    return deg_kernel


def _make_agg(n_rows2d):
    rows_w = n_rows2d // NW
    g_per_w = rows_w // U

    @functools.partial(
        pl.kernel,
        out_type=jax.ShapeDtypeStruct((2, N_PAD, 16), jnp.float32),
        mesh=_mesh,
        compiler_params=pltpu.CompilerParams(use_tc_tiling_on_sc=False),
        scratch_types=[
            pltpu.VMEM_SHARED((N_PAD, 16), jnp.float32),
            pltpu.VMEM((128, 16), jnp.float32),
            pltpu.VMEM((U, EB), jnp.int32),
            pltpu.VMEM((U, EB), jnp.int32),
            pltpu.VMEM((U, EB, 16), jnp.float32),
            pltpu.SemaphoreType.DMA,
            pltpu.SemaphoreType.DMA,
        ],
    )
    def agg_kernel(table, srcp, dstp, out, acc_sh, stage, sidx, didx, rows,
                   sem, sem2):
        c = lax.axis_index("c")
        s = lax.axis_index("s")
        wid = c * 16 + s
        z = jnp.zeros((16,), jnp.float32)

        def zrow(i, carry):
            stage[i, :] = z
            return carry

        lax.fori_loop(0, 128, zrow, 0)

        n_tiles = N_PAD // 128
        q_max = -(-n_tiles // 16)

        def ibody(q, carry):
            t = s + q * 16

            @pl.when(t < n_tiles)
            def _():
                pltpu.sync_copy(stage, acc_sh.at[pl.ds(t * 128, 128)])
            return carry

        lax.fori_loop(0, q_max, ibody, 0)
        plsc.subcore_barrier()

        def body(g, carry):
            row = wid * rows_w + g * U
            ci = [pltpu.async_copy(srcp.at[pl.ds(row, U)], sidx, sem2),
                  pltpu.async_copy(dstp.at[pl.ds(row, U)], didx, sem2)]
            for cp in ci:
                cp.wait()
            h = U // 2
            cpa = [pltpu.async_copy(table.at[sidx.at[j]], rows.at[j], sem)
                   for j in range(h)]
            cpb = [pltpu.async_copy(table.at[sidx.at[j]], rows.at[j], sem)
                   for j in range(h, U)]
            for cp in cpa:
                cp.wait()
            csa = [pltpu.async_copy(rows.at[j], acc_sh.at[didx.at[j]], sem2,
                                    add=True)
                   for j in range(h)]
            for cp in cpb:
                cp.wait()
            csb = [pltpu.async_copy(rows.at[j], acc_sh.at[didx.at[j]], sem2,
                                    add=True)
                   for j in range(h, U)]
            for cp in csa + csb:
                cp.wait()
            return carry

        lax.fori_loop(0, g_per_w, body, 0)
        plsc.subcore_barrier()

        def obody(q, carry):
            t = s + q * 16

            @pl.when(t < n_tiles)
            def _():
                pltpu.sync_copy(acc_sh.at[pl.ds(t * 128, 128)], stage)
                pltpu.sync_copy(stage, out.at[c, pl.ds(t * 128, 128)])
            return carry

        lax.fori_loop(0, q_max, obody, 0)

    return agg_kernel


def _tc_scale1(degs, xpad, W1):
    def body(deg_ref, x_ref, w_ref, dis_ref, h_ref):
        deg = jnp.sum(deg_ref[...], axis=0) + 1.0
        dis = lax.rsqrt(deg)[:, None]
        h = jnp.dot(x_ref[...], w_ref[...], preferred_element_type=jnp.float32)
        dis_ref[...] = dis
        h_ref[...] = h * dis

    return pl.pallas_call(
        body,
        grid=(GRID,),
        in_specs=[
            pl.BlockSpec((2, BN), lambda i: (0, i)),
            pl.BlockSpec((BN, 8), lambda i: (i, 0)),
            pl.BlockSpec((8, 16), lambda i: (0, 0)),
        ],
        out_specs=[
            pl.BlockSpec((BN, 1), lambda i: (i, 0)),
            pl.BlockSpec((BN, 16), lambda i: (i, 0)),
        ],
        out_shape=[
            jax.ShapeDtypeStruct((N_PAD, 1), jnp.float32),
            jax.ShapeDtypeStruct((N_PAD, 16), jnp.float32),
        ],
    )(degs, xpad, W1)


def _tc_mid(acc, h1p, dis, b1, W2p):
    def body(a_ref, h_ref, dis_ref, b_ref, w_ref, o_ref):
        dis = dis_ref[...]
        p = (a_ref[0] + a_ref[1] + h_ref[...]) * dis + b_ref[...]
        r = jnp.maximum(p, 0.0)
        o_ref[...] = jnp.dot(r, w_ref[...],
                             preferred_element_type=jnp.float32) * dis

    return pl.pallas_call(
        body,
        grid=(GRID,),
        in_specs=[
            pl.BlockSpec((2, BN, 16), lambda i: (0, i, 0)),
            pl.BlockSpec((BN, 16), lambda i: (i, 0)),
            pl.BlockSpec((BN, 1), lambda i: (i, 0)),
            pl.BlockSpec((1, 16), lambda i: (0, 0)),
            pl.BlockSpec((16, 16), lambda i: (0, 0)),
        ],
        out_specs=pl.BlockSpec((BN, 16), lambda i: (i, 0)),
        out_shape=jax.ShapeDtypeStruct((N_PAD, 16), jnp.float32),
    )(acc, h1p, dis, b1, W2p)


def _tc_final(acc, h2p, dis, b2p):
    def body(a_ref, h_ref, dis_ref, b_ref, o_ref):
        o_ref[...] = jax.nn.sigmoid(
            (a_ref[0] + a_ref[1] + h_ref[...]) * dis_ref[...] + b_ref[...])

    return pl.pallas_call(
        body,
        grid=(GRID,),
        in_specs=[
            pl.BlockSpec((2, BN, 16), lambda i: (0, i, 0)),
            pl.BlockSpec((BN, 16), lambda i: (i, 0)),
            pl.BlockSpec((BN, 1), lambda i: (i, 0)),
            pl.BlockSpec((1, 16), lambda i: (0, 0)),
        ],
        out_specs=pl.BlockSpec((BN, 16), lambda i: (i, 0)),
        out_shape=jax.ShapeDtypeStruct((N_PAD, 16), jnp.float32),
    )(acc, h2p, dis, b2p)


def kernel(x, edge_index, W1, b1, W2, b2):
    n_edges = edge_index.shape[1]
    chunk = NW * EB * U
    epad = -(-n_edges // chunk) * chunk
    pad = epad - n_edges

    src = edge_index[0].astype(jnp.int32)
    dst = edge_index[1].astype(jnp.int32)
    junk = jnp.full((pad,), N_NODES, jnp.int32)
    srcp = jnp.concatenate([src, junk]).reshape(-1, EB)
    dstp = jnp.concatenate([dst, junk]).reshape(-1, EB)
    n_rows2d = srcp.shape[0]

    xpad = jnp.pad(x, ((0, N_PAD - N_NODES), (0, 0)))
    W2p = jnp.pad(W2, ((0, 0), (0, 16 - W2.shape[1])))
    b1r = b1.reshape(1, 16)
    b2p = jnp.pad(b2, (0, 16 - b2.shape[0])).reshape(1, 16)

    degs = _make_deg(n_rows2d)(dstp).reshape(2, N_PAD)
    dis, h1p = _tc_scale1(degs, xpad, W1)

    agg = _make_agg(n_rows2d)
    acc1 = agg(h1p, srcp, dstp)
    h2p = _tc_mid(acc1, h1p, dis, b1r, W2p)
    acc2 = agg(h2p, srcp, dstp)
    out2 = _tc_final(acc2, h2p, dis, b2p)
    return out2[:N_NODES, :W2.shape[1]]


# speedup vs baseline: 79.0900x; 1.0190x over previous
import functools

import jax
import jax.numpy as jnp
from jax import lax
from jax.experimental import pallas as pl
from jax.experimental.pallas import tpu as pltpu
from jax.experimental.pallas import tpu_sc as plsc

N_NODES = 100000
N_PAD = 100096
STRIPE = N_PAD // 16
CH = STRIPE // 8
EB = 128
U = 12
NW = 32
BN = 5888
GRID = N_PAD // BN

_mesh = plsc.VectorSubcoreMesh(core_axis_name="c", subcore_axis_name="s")


def _make_deg(n_rows2d):
    rows_w = n_rows2d // NW
    g_per_w = rows_w // U

    @functools.partial(
        pl.kernel,
        out_type=jax.ShapeDtypeStruct((2 * N_PAD,), jnp.float32),
        mesh=_mesh,
        compiler_params=pltpu.CompilerParams(use_tc_tiling_on_sc=False),
        scratch_types=[
            pltpu.VMEM_SHARED((N_PAD,), jnp.float32),
            pltpu.VMEM((STRIPE,), jnp.float32),
            pltpu.VMEM((STRIPE,), jnp.float32),
            pltpu.VMEM((U, EB), jnp.int32),
            pltpu.VMEM((EB,), jnp.float32),
            pltpu.SemaphoreType.DMA,
        ],
    )
    def deg_kernel(dstp, out, deg_sh, zbuf, zbuf2, didx, ones_v, sem):
        c = lax.axis_index("c")
        s = lax.axis_index("s")
        wid = c * 16 + s
        z = jnp.zeros((16,), jnp.float32)
        one = jnp.ones((16,), jnp.float32)

        def zbody(i, carry):
            zbuf[pl.ds(i * 16, 16)] = z
            return carry

        lax.fori_loop(0, STRIPE // 16, zbody, 0)
        for k in range(EB // 16):
            ones_v[pl.ds(k * 16, 16)] = one
        pltpu.sync_copy(zbuf, deg_sh.at[pl.ds(s * STRIPE, STRIPE)])
        plsc.subcore_barrier()

        def body(g, carry):
            pltpu.sync_copy(dstp.at[pl.ds(wid * rows_w + g * U, U)], didx)
            cps = [pltpu.async_copy(ones_v, deg_sh.at[didx.at[j]], sem,
                                    add=True)
                   for j in range(U)]
            for cp in cps:
                cp.wait()
            return carry

        lax.fori_loop(0, g_per_w, body, 0)
        plsc.subcore_barrier()
        pltpu.sync_copy(deg_sh.at[pl.ds(s * STRIPE, STRIPE)], zbuf)

        def cbody(i, carry):
            zbuf2[pl.ds(i * 16, 16)] = zbuf[pl.ds(i * 16, 16)]
            return carry

        lax.fori_loop(0, STRIPE // 16, cbody, 0)
        pltpu.sync_copy(zbuf2, out.at[pl.ds(c * N_PAD + s * STRIPE, STRIPE)])

    return deg_kernel


def _make_agg(n_rows2d):
    rows_w = n_rows2d // NW
    g_per_w = rows_w // U

    @functools.partial(
        pl.kernel,
        out_type=jax.ShapeDtypeStruct((2, N_PAD, 16), jnp.float32),
        mesh=_mesh,
        compiler_params=pltpu.CompilerParams(use_tc_tiling_on_sc=False),
        scratch_types=[
            pltpu.VMEM_SHARED((N_PAD, 16), jnp.float32),
            pltpu.VMEM((128, 16), jnp.float32),
            pltpu.VMEM((U, EB), jnp.int32),
            pltpu.VMEM((U, EB), jnp.int32),
            pltpu.VMEM((U, EB, 16), jnp.float32),
            pltpu.SemaphoreType.DMA,
            pltpu.SemaphoreType.DMA,
        ],
    )
    def agg_kernel(table, srcp, dstp, out, acc_sh, stage, sidx, didx, rows,
                   sem, sem2):
        c = lax.axis_index("c")
        s = lax.axis_index("s")
        wid = c * 16 + s
        z = jnp.zeros((16,), jnp.float32)

        def zrow(i, carry):
            stage[i, :] = z
            return carry

        lax.fori_loop(0, 128, zrow, 0)

        n_tiles = N_PAD // 128
        q_max = -(-n_tiles // 16)

        def ibody(q, carry):
            t = s + q * 16

            @pl.when(t < n_tiles)
            def _():
                pltpu.sync_copy(stage, acc_sh.at[pl.ds(t * 128, 128)])
            return carry

        lax.fori_loop(0, q_max, ibody, 0)
        plsc.subcore_barrier()

        def body(g, carry):
            row = wid * rows_w + g * U
            ci = [pltpu.async_copy(srcp.at[pl.ds(row, U)], sidx, sem2),
                  pltpu.async_copy(dstp.at[pl.ds(row, U)], didx, sem2)]
            for cp in ci:
                cp.wait()
            h = U // 3
            parts = [range(0, h), range(h, 2 * h), range(2 * h, U)]
            gth = [[pltpu.async_copy(table.at[sidx.at[j]], rows.at[j], sem)
                    for j in p] for p in parts]
            sct = []
            for gp, p in zip(gth, parts):
                for cp in gp:
                    cp.wait()
                sct += [pltpu.async_copy(rows.at[j], acc_sh.at[didx.at[j]],
                                         sem2, add=True)
                        for j in p]
            for cp in sct:
                cp.wait()
            return carry

        lax.fori_loop(0, g_per_w, body, 0)
        plsc.subcore_barrier()

        def obody(q, carry):
            t = s + q * 16

            @pl.when(t < n_tiles)
            def _():
                pltpu.sync_copy(acc_sh.at[pl.ds(t * 128, 128)], stage)
                pltpu.sync_copy(stage, out.at[c, pl.ds(t * 128, 128)])
            return carry

        lax.fori_loop(0, q_max, obody, 0)

    return agg_kernel


def _tc_scale1(degs, xpad, W1):
    def body(deg_ref, x_ref, w_ref, dis_ref, h_ref):
        deg = jnp.sum(deg_ref[...], axis=0) + 1.0
        dis = lax.rsqrt(deg)[:, None]
        h = jnp.dot(x_ref[...], w_ref[...], preferred_element_type=jnp.float32)
        dis_ref[...] = dis
        h_ref[...] = h * dis

    return pl.pallas_call(
        body,
        grid=(GRID,),
        in_specs=[
            pl.BlockSpec((2, BN), lambda i: (0, i)),
            pl.BlockSpec((BN, 8), lambda i: (i, 0)),
            pl.BlockSpec((8, 16), lambda i: (0, 0)),
        ],
        out_specs=[
            pl.BlockSpec((BN, 1), lambda i: (i, 0)),
            pl.BlockSpec((BN, 16), lambda i: (i, 0)),
        ],
        out_shape=[
            jax.ShapeDtypeStruct((N_PAD, 1), jnp.float32),
            jax.ShapeDtypeStruct((N_PAD, 16), jnp.float32),
        ],
    )(degs, xpad, W1)


def _tc_mid(acc, h1p, dis, b1, W2p):
    def body(a_ref, h_ref, dis_ref, b_ref, w_ref, o_ref):
        dis = dis_ref[...]
        p = (a_ref[0] + a_ref[1] + h_ref[...]) * dis + b_ref[...]
        r = jnp.maximum(p, 0.0)
        o_ref[...] = jnp.dot(r, w_ref[...],
                             preferred_element_type=jnp.float32) * dis

    return pl.pallas_call(
        body,
        grid=(GRID,),
        in_specs=[
            pl.BlockSpec((2, BN, 16), lambda i: (0, i, 0)),
            pl.BlockSpec((BN, 16), lambda i: (i, 0)),
            pl.BlockSpec((BN, 1), lambda i: (i, 0)),
            pl.BlockSpec((1, 16), lambda i: (0, 0)),
            pl.BlockSpec((16, 16), lambda i: (0, 0)),
        ],
        out_specs=pl.BlockSpec((BN, 16), lambda i: (i, 0)),
        out_shape=jax.ShapeDtypeStruct((N_PAD, 16), jnp.float32),
    )(acc, h1p, dis, b1, W2p)


def _tc_final(acc, h2p, dis, b2p):
    def body(a_ref, h_ref, dis_ref, b_ref, o_ref):
        o_ref[...] = jax.nn.sigmoid(
            (a_ref[0] + a_ref[1] + h_ref[...]) * dis_ref[...] + b_ref[...])

    return pl.pallas_call(
        body,
        grid=(GRID,),
        in_specs=[
            pl.BlockSpec((2, BN, 16), lambda i: (0, i, 0)),
            pl.BlockSpec((BN, 16), lambda i: (i, 0)),
            pl.BlockSpec((BN, 1), lambda i: (i, 0)),
            pl.BlockSpec((1, 16), lambda i: (0, 0)),
        ],
        out_specs=pl.BlockSpec((BN, 16), lambda i: (i, 0)),
        out_shape=jax.ShapeDtypeStruct((N_PAD, 16), jnp.float32),
    )(acc, h2p, dis, b2p)


def kernel(x, edge_index, W1, b1, W2, b2):
    n_edges = edge_index.shape[1]
    chunk = NW * EB * U
    epad = -(-n_edges // chunk) * chunk
    pad = epad - n_edges

    src = edge_index[0].astype(jnp.int32)
    dst = edge_index[1].astype(jnp.int32)
    junk = jnp.full((pad,), N_NODES, jnp.int32)
    srcp = jnp.concatenate([src, junk]).reshape(-1, EB)
    dstp = jnp.concatenate([dst, junk]).reshape(-1, EB)
    n_rows2d = srcp.shape[0]

    xpad = jnp.pad(x, ((0, N_PAD - N_NODES), (0, 0)))
    W2p = jnp.pad(W2, ((0, 0), (0, 16 - W2.shape[1])))
    b1r = b1.reshape(1, 16)
    b2p = jnp.pad(b2, (0, 16 - b2.shape[0])).reshape(1, 16)

    degs = _make_deg(n_rows2d)(dstp).reshape(2, N_PAD)
    dis, h1p = _tc_scale1(degs, xpad, W1)

    agg = _make_agg(n_rows2d)
    acc1 = agg(h1p, srcp, dstp)
    h2p = _tc_mid(acc1, h1p, dis, b1r, W2p)
    acc2 = agg(h2p, srcp, dstp)
    out2 = _tc_final(acc2, h2p, dis, b2p)
    return out2[:N_NODES, :W2.shape[1]]


# speedup vs baseline: 79.8921x; 1.0101x over previous
import functools

import jax
import jax.numpy as jnp
from jax import lax
from jax.experimental import pallas as pl
from jax.experimental.pallas import tpu as pltpu
from jax.experimental.pallas import tpu_sc as plsc

N_NODES = 100000
N_PAD = 100096
STRIPE = N_PAD // 16
CH = STRIPE // 8
EB = 128
U = 12
NW = 32
BN = 5888
GRID = N_PAD // BN

_mesh = plsc.VectorSubcoreMesh(core_axis_name="c", subcore_axis_name="s")


def _make_deg(n_rows2d):
    rows_w = n_rows2d // NW
    g_per_w = rows_w // U

    @functools.partial(
        pl.kernel,
        out_type=jax.ShapeDtypeStruct((2 * N_PAD,), jnp.float32),
        mesh=_mesh,
        compiler_params=pltpu.CompilerParams(use_tc_tiling_on_sc=False),
        scratch_types=[
            pltpu.VMEM_SHARED((N_PAD,), jnp.float32),
            pltpu.VMEM((STRIPE,), jnp.float32),
            pltpu.VMEM((STRIPE,), jnp.float32),
            pltpu.VMEM((U, EB), jnp.int32),
            pltpu.VMEM((EB,), jnp.float32),
            pltpu.SemaphoreType.DMA,
        ],
    )
    def deg_kernel(dstp, out, deg_sh, zbuf, zbuf2, didx, ones_v, sem):
        c = lax.axis_index("c")
        s = lax.axis_index("s")
        wid = c * 16 + s
        z = jnp.zeros((16,), jnp.float32)
        one = jnp.ones((16,), jnp.float32)

        def zbody(i, carry):
            zbuf[pl.ds(i * 16, 16)] = z
            return carry

        lax.fori_loop(0, STRIPE // 16, zbody, 0)
        for k in range(EB // 16):
            ones_v[pl.ds(k * 16, 16)] = one
        pltpu.sync_copy(zbuf, deg_sh.at[pl.ds(s * STRIPE, STRIPE)])
        plsc.subcore_barrier()

        def body(g, carry):
            pltpu.sync_copy(dstp.at[pl.ds(wid * rows_w + g * U, U)], didx)
            cps = [pltpu.async_copy(ones_v, deg_sh.at[didx.at[j]], sem,
                                    add=True)
                   for j in range(U)]
            for cp in cps:
                cp.wait()
            return carry

        lax.fori_loop(0, g_per_w, body, 0)
        plsc.subcore_barrier()
        pltpu.sync_copy(deg_sh.at[pl.ds(s * STRIPE, STRIPE)], zbuf)

        def cbody(i, carry):
            zbuf2[pl.ds(i * 16, 16)] = zbuf[pl.ds(i * 16, 16)]
            return carry

        lax.fori_loop(0, STRIPE // 16, cbody, 0)
        pltpu.sync_copy(zbuf2, out.at[pl.ds(c * N_PAD + s * STRIPE, STRIPE)])

    return deg_kernel


def _make_agg(n_rows2d):
    rows_w = n_rows2d // NW
    g_per_w = rows_w // U

    @functools.partial(
        pl.kernel,
        out_type=jax.ShapeDtypeStruct((2, N_PAD, 16), jnp.float32),
        mesh=_mesh,
        compiler_params=pltpu.CompilerParams(use_tc_tiling_on_sc=False),
        scratch_types=[
            pltpu.VMEM_SHARED((N_PAD, 16), jnp.float32),
            pltpu.VMEM((128, 16), jnp.float32),
            pltpu.VMEM((U, EB), jnp.int32),
            pltpu.VMEM((U, EB), jnp.int32),
            pltpu.VMEM((U, EB, 16), jnp.float32),
            pltpu.SemaphoreType.DMA,
            pltpu.SemaphoreType.DMA,
        ],
    )
    def agg_kernel(table, srcp, dstp, out, acc_sh, stage, sidx, didx, rows,
                   sem, sem2):
        c = lax.axis_index("c")
        s = lax.axis_index("s")
        wid = c * 16 + s
        z = jnp.zeros((16,), jnp.float32)

        def zrow(i, carry):
            stage[i, :] = z
            return carry

        lax.fori_loop(0, 128, zrow, 0)

        n_tiles = N_PAD // 128
        q_max = -(-n_tiles // 16)

        def ibody(q, carry):
            t = s + q * 16

            @pl.when(t < n_tiles)
            def _():
                pltpu.sync_copy(stage, acc_sh.at[pl.ds(t * 128, 128)])
            return carry

        lax.fori_loop(0, q_max, ibody, 0)
        plsc.subcore_barrier()

        def body(g, carry):
            row = wid * rows_w + g * U
            ci = [pltpu.async_copy(srcp.at[pl.ds(row, U)], sidx, sem2),
                  pltpu.async_copy(dstp.at[pl.ds(row, U)], didx, sem2)]
            for cp in ci:
                cp.wait()
            h = U // 4
            parts = [range(k * h, (k + 1) * h) for k in range(4)]
            gth = [[pltpu.async_copy(table.at[sidx.at[j]], rows.at[j], sem)
                    for j in p] for p in parts]
            sct = []
            for gp, p in zip(gth, parts):
                for cp in gp:
                    cp.wait()
                sct += [pltpu.async_copy(rows.at[j], acc_sh.at[didx.at[j]],
                                         sem2, add=True)
                        for j in p]
            for cp in sct:
                cp.wait()
            return carry

        lax.fori_loop(0, g_per_w, body, 0)
        plsc.subcore_barrier()

        def obody(q, carry):
            t = s + q * 16

            @pl.when(t < n_tiles)
            def _():
                pltpu.sync_copy(acc_sh.at[pl.ds(t * 128, 128)], stage)
                pltpu.sync_copy(stage, out.at[c, pl.ds(t * 128, 128)])
            return carry

        lax.fori_loop(0, q_max, obody, 0)

    return agg_kernel


def _tc_scale1(degs, xpad, W1):
    def body(deg_ref, x_ref, w_ref, dis_ref, h_ref):
        deg = jnp.sum(deg_ref[...], axis=0) + 1.0
        dis = lax.rsqrt(deg)[:, None]
        h = jnp.dot(x_ref[...], w_ref[...], preferred_element_type=jnp.float32)
        dis_ref[...] = dis
        h_ref[...] = h * dis

    return pl.pallas_call(
        body,
        grid=(GRID,),
        in_specs=[
            pl.BlockSpec((2, BN), lambda i: (0, i)),
            pl.BlockSpec((BN, 8), lambda i: (i, 0)),
            pl.BlockSpec((8, 16), lambda i: (0, 0)),
        ],
        out_specs=[
            pl.BlockSpec((BN, 1), lambda i: (i, 0)),
            pl.BlockSpec((BN, 16), lambda i: (i, 0)),
        ],
        out_shape=[
            jax.ShapeDtypeStruct((N_PAD, 1), jnp.float32),
            jax.ShapeDtypeStruct((N_PAD, 16), jnp.float32),
        ],
    )(degs, xpad, W1)


def _tc_mid(acc, h1p, dis, b1, W2p):
    def body(a_ref, h_ref, dis_ref, b_ref, w_ref, o_ref):
        dis = dis_ref[...]
        p = (a_ref[0] + a_ref[1] + h_ref[...]) * dis + b_ref[...]
        r = jnp.maximum(p, 0.0)
        o_ref[...] = jnp.dot(r, w_ref[...],
                             preferred_element_type=jnp.float32) * dis

    return pl.pallas_call(
        body,
        grid=(GRID,),
        in_specs=[
            pl.BlockSpec((2, BN, 16), lambda i: (0, i, 0)),
            pl.BlockSpec((BN, 16), lambda i: (i, 0)),
            pl.BlockSpec((BN, 1), lambda i: (i, 0)),
            pl.BlockSpec((1, 16), lambda i: (0, 0)),
            pl.BlockSpec((16, 16), lambda i: (0, 0)),
        ],
        out_specs=pl.BlockSpec((BN, 16), lambda i: (i, 0)),
        out_shape=jax.ShapeDtypeStruct((N_PAD, 16), jnp.float32),
    )(acc, h1p, dis, b1, W2p)


def _tc_final(acc, h2p, dis, b2p):
    def body(a_ref, h_ref, dis_ref, b_ref, o_ref):
        o_ref[...] = jax.nn.sigmoid(
            (a_ref[0] + a_ref[1] + h_ref[...]) * dis_ref[...] + b_ref[...])

    return pl.pallas_call(
        body,
        grid=(GRID,),
        in_specs=[
            pl.BlockSpec((2, BN, 16), lambda i: (0, i, 0)),
            pl.BlockSpec((BN, 16), lambda i: (i, 0)),
            pl.BlockSpec((BN, 1), lambda i: (i, 0)),
            pl.BlockSpec((1, 16), lambda i: (0, 0)),
        ],
        out_specs=pl.BlockSpec((BN, 16), lambda i: (i, 0)),
        out_shape=jax.ShapeDtypeStruct((N_PAD, 16), jnp.float32),
    )(acc, h2p, dis, b2p)


def kernel(x, edge_index, W1, b1, W2, b2):
    n_edges = edge_index.shape[1]
    chunk = NW * EB * U
    epad = -(-n_edges // chunk) * chunk
    pad = epad - n_edges

    src = edge_index[0].astype(jnp.int32)
    dst = edge_index[1].astype(jnp.int32)
    junk = jnp.full((pad,), N_NODES, jnp.int32)
    srcp = jnp.concatenate([src, junk]).reshape(-1, EB)
    dstp = jnp.concatenate([dst, junk]).reshape(-1, EB)
    n_rows2d = srcp.shape[0]

    xpad = jnp.pad(x, ((0, N_PAD - N_NODES), (0, 0)))
    W2p = jnp.pad(W2, ((0, 0), (0, 16 - W2.shape[1])))
    b1r = b1.reshape(1, 16)
    b2p = jnp.pad(b2, (0, 16 - b2.shape[0])).reshape(1, 16)

    degs = _make_deg(n_rows2d)(dstp).reshape(2, N_PAD)
    dis, h1p = _tc_scale1(degs, xpad, W1)

    agg = _make_agg(n_rows2d)
    acc1 = agg(h1p, srcp, dstp)
    h2p = _tc_mid(acc1, h1p, dis, b1r, W2p)
    acc2 = agg(h2p, srcp, dstp)
    out2 = _tc_final(acc2, h2p, dis, b2p)
    return out2[:N_NODES, :W2.shape[1]]
